# Initial kernel scaffold; baseline (speedup 1.0000x reference)
import jax
import jax.numpy as jnp
from jax.experimental import pallas as pl


def kernel(features, vertices, faces, enc_W1, enc_b1, enc_W2, enc_b2, g1_W, g1_b, g2_W, g2_b, g3_W, g3_b, g4_W, g4_b, g5_W, g5_b, g6_W, g6_b, h1_W, h1_b, h2_W, h2_b, h3_W, h3_b):
    raise NotImplementedError("write your pallas kernel here")


# speedup vs baseline: 2.1024x; 2.1024x over previous
import functools

import jax
import jax.numpy as jnp
from jax import lax
from jax.experimental import pallas as pl
from jax.experimental.pallas import tpu as pltpu
from jax.experimental.pallas import tpu_sc as plsc

N_V = 10000
NP = 10240
NB = 2
M_ROWS = NB * NP
N_E = 60000
E_PAD = 122880
ZROW = N_V


def _mm_y(x, w, dis128):
    M, K = x.shape
    Nout = w.shape[1]
    BM, BN = 256, min(256, Nout)
    def body(x_ref, w_ref, d_ref, o_ref):
        acc = jnp.dot(x_ref[...], w_ref[...],
                      preferred_element_type=jnp.float32)
        o_ref[...] = acc * d_ref[:, 0:1]
    return pl.pallas_call(
        body,
        grid=(M // BM, Nout // BN),
        in_specs=[pl.BlockSpec((BM, K), lambda i, j: (i, 0)),
                  pl.BlockSpec((K, BN), lambda i, j: (0, j)),
                  pl.BlockSpec((BM, 128), lambda i, j: (i, 0))],
        out_specs=pl.BlockSpec((BM, BN), lambda i, j: (i, j)),
        out_shape=jax.ShapeDtypeStruct((M, Nout), jnp.float32),
    )(x, w, dis128)


def _mm_bias_act(x, w, b8, act):
    M, K = x.shape
    Nout = w.shape[1]
    BM, BN = 256, 128
    def body(x_ref, w_ref, b_ref, o_ref):
        acc = jnp.dot(x_ref[...], w_ref[...],
                      preferred_element_type=jnp.float32)
        acc = acc + b_ref[0:1, :]
        if act == 'relu':
            acc = jnp.maximum(acc, 0.0)
        elif act == 'tanh':
            acc = jnp.tanh(acc)
        o_ref[...] = acc
    return pl.pallas_call(
        body,
        grid=(M // BM, Nout // BN),
        in_specs=[pl.BlockSpec((BM, K), lambda i, j: (i, 0)),
                  pl.BlockSpec((K, BN), lambda i, j: (0, j)),
                  pl.BlockSpec((8, BN), lambda i, j: (0, j))],
        out_specs=pl.BlockSpec((BM, BN), lambda i, j: (i, j)),
        out_shape=jax.ShapeDtypeStruct((M, Nout), jnp.float32),
    )(x, w, b8)


def _cumsum_excl(z):
    E, D = z.shape
    RB = 1024
    CB = min(D, 256)
    def body(z_ref, o_ref, carry):
        r = pl.program_id(1)
        @pl.when(r == 0)
        def _():
            carry[...] = jnp.zeros_like(carry)
        ics = z_ref[...]
        sh = 1
        while sh < RB:
            ics = ics + jnp.concatenate(
                [jnp.zeros((sh, CB), jnp.float32), ics[:-sh]], axis=0)
            sh *= 2
        c = carry[0:1, :]
        o_ref[...] = c + jnp.concatenate(
            [jnp.zeros((1, CB), jnp.float32), ics[:-1]], axis=0)
        carry[0:1, :] = c + ics[-1:]
    return pl.pallas_call(
        body,
        grid=(D // CB, E // RB),
        in_specs=[pl.BlockSpec((RB, CB), lambda j, r: (r, j))],
        out_specs=pl.BlockSpec((RB, CB), lambda j, r: (r, j)),
        out_shape=jax.ShapeDtypeStruct((E, D), jnp.float32),
        scratch_shapes=[pltpu.VMEM((8, CB), jnp.float32)],
        compiler_params=pltpu.CompilerParams(
            dimension_semantics=("arbitrary", "arbitrary")),
    )(z)


def _combine(gE, gS, y, dis128, b8):
    M, D = y.shape
    BM = 256
    CB = min(D, 256)
    def body(e_ref, s_ref, y_ref, d_ref, b_ref, o_ref):
        agg = e_ref[...] - s_ref[...] + y_ref[...]
        o_ref[...] = jnp.maximum(d_ref[:, 0:1] * agg + b_ref[0:1, :], 0.0)
    return pl.pallas_call(
        body,
        grid=(M // BM, D // CB),
        in_specs=[pl.BlockSpec((BM, CB), lambda i, j: (i, j)),
                  pl.BlockSpec((BM, CB), lambda i, j: (i, j)),
                  pl.BlockSpec((BM, CB), lambda i, j: (i, j)),
                  pl.BlockSpec((BM, 128), lambda i, j: (i, 0)),
                  pl.BlockSpec((8, CB), lambda i, j: (0, j))],
        out_specs=pl.BlockSpec((BM, CB), lambda i, j: (i, j)),
        out_shape=jax.ShapeDtypeStruct((M, D), jnp.float32),
    )(gE, gS, y, dis128, b8)


def _gather_rows(table, idx):
    V, D = table.shape
    Bi = idx.shape[0]
    info = plsc.get_sparse_core_info()
    NW = info.num_cores * info.num_subcores
    bpw = Bi // NW
    assert Bi % NW == 0 and bpw % 8 == 0
    ch = None
    for c in range(min(bpw, max(8, 98304 // D)), 7, -8):
        if bpw % c == 0:
            ch = c
            break
    assert ch is not None
    nch = bpw // ch
    mesh = plsc.VectorSubcoreMesh(core_axis_name="c", subcore_axis_name="s")

    @functools.partial(
        pl.kernel, mesh=mesh,
        out_type=jax.ShapeDtypeStruct((Bi, D), jnp.float32),
        scratch_types=[pltpu.VMEM((ch,), jnp.int32),
                       pltpu.VMEM((ch, D), jnp.float32),
                       pltpu.SemaphoreType.DMA])
    def k(table_hbm, idx_hbm, out_hbm, idx_v, rows_v, sem):
        wid = lax.axis_index("s") * info.num_cores + lax.axis_index("c")
        base0 = wid * bpw
        def body(i, carry):
            base = base0 + i * ch
            pltpu.sync_copy(idx_hbm.at[pl.ds(base, ch)], idx_v)
            pltpu.async_copy(table_hbm.at[idx_v], rows_v, sem).wait()
            pltpu.sync_copy(rows_v, out_hbm.at[pl.ds(base, ch)])
            return carry
        lax.fori_loop(0, nch, body, 0)

    return k(table, idx)


def _trilinear_const(features):
    Bn, C, D, H, W = features.shape
    g = jnp.float32(-1.0 / 1.5)
    ix = jnp.clip((g + 1.0) * 0.5 * (W - 1), 0.0, W - 1.0)
    x0 = jnp.floor(ix)
    w1 = (ix - x0).astype(jnp.float32)
    w0 = 1.0 - w1
    i0 = jnp.clip(x0.astype(jnp.int32), 0, W - 1)
    i1 = jnp.clip(i0 + 1, 0, W - 1)
    idx = jnp.stack([i0, i1])
    wts = jnp.stack([w0, w1])
    vf = jnp.zeros((Bn, C), jnp.float32)
    for zi in range(2):
        for yi in range(2):
            for xi in range(2):
                corner = jax.vmap(
                    lambda f: lax.dynamic_index_in_dim(
                        lax.dynamic_index_in_dim(
                            lax.dynamic_index_in_dim(
                                f, idx[zi], axis=1, keepdims=False),
                            idx[yi], axis=1, keepdims=False),
                        idx[xi], axis=1, keepdims=False))(features)
                vf = vf + corner * (wts[zi] * wts[yi] * wts[xi])
    return vf


def _pad_cols(a, n):
    return jnp.pad(a, ((0, 0), (0, n - a.shape[1])))


def _pad_rows(a, n):
    return jnp.pad(a, ((0, n - a.shape[0]), (0, 0)))


def kernel(features, vertices, faces, enc_W1, enc_b1, enc_W2, enc_b2,
           g1_W, g1_b, g2_W, g2_b, g3_W, g3_b, g4_W, g4_b, g5_W, g5_b,
           g6_W, g6_b, h1_W, h1_b, h2_W, h2_b, h3_W, h3_b):
    f32 = jnp.float32

    e = jnp.concatenate([faces[:, 0:2], faces[:, 1:3],
                         faces[:, jnp.array([0, 2])]], axis=0)
    key = e[:, 0] * jnp.int32(N_V) + e[:, 1]
    order = jnp.argsort(key)
    es = e[order]
    ks = key[order]
    first = jnp.concatenate([jnp.ones((1,), jnp.bool_), ks[1:] != ks[:-1]])
    emask = first.astype(f32)
    src, dst = es[:, 0], es[:, 1]
    order2 = jnp.argsort(dst)
    src_d = src[order2]
    dst_d = dst[order2]
    m_d = emask[order2]
    off = jnp.searchsorted(dst_d, jnp.arange(N_V + 1, dtype=jnp.int32),
                           side='left').astype(jnp.int32)
    cm = jnp.concatenate([jnp.zeros((1,), f32), jnp.cumsum(m_d)])
    deg = (cm[off[1:]] - cm[off[:-1]]) + 1.0
    dis_n = lax.rsqrt(deg)

    dis = jnp.zeros((M_ROWS,), f32)
    dis = dis.at[0:N_V].set(dis_n).at[NP:NP + N_V].set(dis_n)
    dis128 = jnp.broadcast_to(dis[:, None], (M_ROWS, 128))

    srcg = jnp.where(m_d > 0, src_d, ZROW)
    src2 = jnp.concatenate([
        srcg, srcg + NP,
        jnp.full((E_PAD - NB * N_E,), ZROW, jnp.int32)]).astype(jnp.int32)
    pad_idx = jnp.zeros((NP - N_V,), jnp.int32)
    idxS = jnp.concatenate([off[:-1], pad_idx,
                            N_E + off[:-1], pad_idx]).astype(jnp.int32)
    idxE = jnp.concatenate([off[1:], pad_idx,
                            N_E + off[1:], pad_idx]).astype(jnp.int32)

    vf = _trilinear_const(features)
    xb = jnp.concatenate(
        [jnp.broadcast_to(vertices[None], (NB, N_V, 3)),
         jnp.broadcast_to(vf[:, None, :], (NB, N_V, vf.shape[1]))], axis=-1)
    x0 = jnp.zeros((NB, NP, 256), f32)
    x0 = x0.at[:, :N_V, :3 + vf.shape[1]].set(xb).reshape(M_ROWS, 256)

    x = _mm_bias_act(x0, _pad_rows(enc_W1, 256),
                     jnp.broadcast_to(enc_b1[None], (8, 256)), 'relu')
    x = _mm_bias_act(x, enc_W2,
                     jnp.broadcast_to(enc_b2[None], (8, 128)), 'relu')

    for W, b in ((g1_W, g1_b), (g2_W, g2_b), (g3_W, g3_b),
                 (g4_W, g4_b), (g5_W, g5_b), (g6_W, g6_b)):
        fo = W.shape[1]
        y = _mm_y(x, W, dis128)
        z = _gather_rows(y, src2)
        csx = _cumsum_excl(z)
        gE = _gather_rows(csx, idxE)
        gS = _gather_rows(csx, idxS)
        x = _combine(gE, gS, y, dis128,
                     jnp.broadcast_to(b[None], (8, fo)))

    x = _mm_bias_act(x, h1_W, jnp.broadcast_to(h1_b[None], (8, 128)), 'relu')
    x = _mm_bias_act(x, _pad_cols(h2_W, 128),
                     jnp.broadcast_to(jnp.pad(h2_b, (0, 64))[None], (8, 128)),
                     'relu')
    x = _mm_bias_act(x, _pad_cols(_pad_rows(h3_W, 128), 128),
                     jnp.broadcast_to(jnp.pad(h3_b, (0, 125))[None], (8, 128)),
                     'tanh')

    disp = jnp.stack([x[0:N_V, 0:3], x[NP:NP + N_V, 0:3]])
    disp = jnp.nan_to_num(disp, nan=0.0)
    return vertices + jnp.clip(disp, -2.5, 2.5)


# speedup vs baseline: 2.1999x; 1.0464x over previous
import functools

import jax
import jax.numpy as jnp
from jax import lax
from jax.experimental import pallas as pl
from jax.experimental.pallas import tpu as pltpu
from jax.experimental.pallas import tpu_sc as plsc

N_V = 10000
NP = 10240
NB = 2
M_ROWS = NB * NP
N_E = 60000
E_PAD = 122880
ZROW = N_V


def _mm_y(x, w, dis128):
    M, K = x.shape
    Nout = w.shape[1]
    BM, BN = 256, min(256, Nout)
    def body(x_ref, w_ref, d_ref, o_ref):
        acc = jnp.dot(x_ref[...], w_ref[...],
                      preferred_element_type=jnp.float32)
        o_ref[...] = acc * d_ref[:, 0:1]
    return pl.pallas_call(
        body,
        grid=(M // BM, Nout // BN),
        in_specs=[pl.BlockSpec((BM, K), lambda i, j: (i, 0)),
                  pl.BlockSpec((K, BN), lambda i, j: (0, j)),
                  pl.BlockSpec((BM, 128), lambda i, j: (i, 0))],
        out_specs=pl.BlockSpec((BM, BN), lambda i, j: (i, j)),
        out_shape=jax.ShapeDtypeStruct((M, Nout), jnp.float32),
    )(x, w, dis128)


def _mm_bias_act(x, w, b8, act):
    M, K = x.shape
    Nout = w.shape[1]
    BM, BN = 256, 128
    def body(x_ref, w_ref, b_ref, o_ref):
        acc = jnp.dot(x_ref[...], w_ref[...],
                      preferred_element_type=jnp.float32)
        acc = acc + b_ref[0:1, :]
        if act == 'relu':
            acc = jnp.maximum(acc, 0.0)
        elif act == 'tanh':
            acc = jnp.tanh(acc)
        o_ref[...] = acc
    return pl.pallas_call(
        body,
        grid=(M // BM, Nout // BN),
        in_specs=[pl.BlockSpec((BM, K), lambda i, j: (i, 0)),
                  pl.BlockSpec((K, BN), lambda i, j: (0, j)),
                  pl.BlockSpec((8, BN), lambda i, j: (0, j))],
        out_specs=pl.BlockSpec((BM, BN), lambda i, j: (i, j)),
        out_shape=jax.ShapeDtypeStruct((M, Nout), jnp.float32),
    )(x, w, b8)


def _cumsum_excl(z):
    E, D = z.shape
    RB = 1024
    CB = min(D, 256)
    def body(z_ref, o_ref, carry):
        r = pl.program_id(1)
        @pl.when(r == 0)
        def _():
            carry[...] = jnp.zeros_like(carry)
        ics = z_ref[...]
        sh = 1
        while sh < RB:
            ics = ics + jnp.concatenate(
                [jnp.zeros((sh, CB), jnp.float32), ics[:-sh]], axis=0)
            sh *= 2
        c = carry[0:1, :]
        o_ref[...] = c + jnp.concatenate(
            [jnp.zeros((1, CB), jnp.float32), ics[:-1]], axis=0)
        carry[0:1, :] = c + ics[-1:]
    return pl.pallas_call(
        body,
        grid=(D // CB, E // RB),
        in_specs=[pl.BlockSpec((RB, CB), lambda j, r: (r, j))],
        out_specs=pl.BlockSpec((RB, CB), lambda j, r: (r, j)),
        out_shape=jax.ShapeDtypeStruct((E, D), jnp.float32),
        scratch_shapes=[pltpu.VMEM((8, CB), jnp.float32)],
        compiler_params=pltpu.CompilerParams(
            dimension_semantics=("arbitrary", "arbitrary")),
    )(z)


def _combine(gES, y, dis128, b8):
    M, D = y.shape
    BM = 256
    CB = min(D, 256)
    nb = M // BM
    def body(e_ref, s_ref, y_ref, d_ref, b_ref, o_ref):
        agg = e_ref[...] - s_ref[...] + y_ref[...]
        o_ref[...] = jnp.maximum(d_ref[:, 0:1] * agg + b_ref[0:1, :], 0.0)
    return pl.pallas_call(
        body,
        grid=(nb, D // CB),
        in_specs=[pl.BlockSpec((BM, CB), lambda i, j: (i, j)),
                  pl.BlockSpec((BM, CB), lambda i, j: (i + nb, j)),
                  pl.BlockSpec((BM, CB), lambda i, j: (i, j)),
                  pl.BlockSpec((BM, 128), lambda i, j: (i, 0)),
                  pl.BlockSpec((8, CB), lambda i, j: (0, j))],
        out_specs=pl.BlockSpec((BM, CB), lambda i, j: (i, j)),
        out_shape=jax.ShapeDtypeStruct((M, D), jnp.float32),
    )(gES, gES, y, dis128, b8)


def _gather_rows(table, idx):
    V, D = table.shape
    Bi = idx.shape[0]
    info = plsc.get_sparse_core_info()
    NW = info.num_cores * info.num_subcores
    bpw = Bi // NW
    assert Bi % NW == 0 and bpw % 8 == 0
    ch = 8
    for c in range(min(bpw, max(8, 49152 // D)), 7, -8):
        if bpw % c == 0 and (bpw // c) % 2 == 0:
            ch = c
            break
    nch = bpw // ch
    assert nch % 2 == 0
    mesh = plsc.VectorSubcoreMesh(core_axis_name="c", subcore_axis_name="s")

    @functools.partial(
        pl.kernel, mesh=mesh,
        out_type=jax.ShapeDtypeStruct((Bi, D), jnp.float32),
        scratch_types=[pltpu.VMEM((ch,), jnp.int32),
                       pltpu.VMEM((ch,), jnp.int32),
                       pltpu.VMEM((ch, D), jnp.float32),
                       pltpu.VMEM((ch, D), jnp.float32),
                       pltpu.SemaphoreType.DMA,
                       pltpu.SemaphoreType.DMA])
    def k(table_hbm, idx_hbm, out_hbm, idx0, idx1, rows0, rows1, g0, g1):
        wid = lax.axis_index("s") * info.num_cores + lax.axis_index("c")
        base0 = wid * bpw

        def start(iv, rv, sem, j):
            pltpu.sync_copy(idx_hbm.at[pl.ds(base0 + j * ch, ch)], iv)
            pltpu.async_copy(table_hbm.at[iv], rv, sem)

        def finish(iv, rv, sem, j):
            pltpu.make_async_copy(table_hbm.at[iv], rv, sem).wait()
            pltpu.sync_copy(rv, out_hbm.at[pl.ds(base0 + j * ch, ch)])

        start(idx0, rows0, g0, 0)

        def body(j2, carry):
            a = 2 * j2
            start(idx1, rows1, g1, a + 1)
            finish(idx0, rows0, g0, a)
            @pl.when(j2 + 1 < nch // 2)
            def _():
                start(idx0, rows0, g0, a + 2)
            finish(idx1, rows1, g1, a + 1)
            return carry
        lax.fori_loop(0, nch // 2, body, 0)

    return k(table, idx)


def _trilinear_const(features):
    Bn, C, D, H, W = features.shape
    g = jnp.float32(-1.0 / 1.5)
    ix = jnp.clip((g + 1.0) * 0.5 * (W - 1), 0.0, W - 1.0)
    x0 = jnp.floor(ix)
    w1 = (ix - x0).astype(jnp.float32)
    w0 = 1.0 - w1
    i0 = jnp.clip(x0.astype(jnp.int32), 0, W - 1)
    i1 = jnp.clip(i0 + 1, 0, W - 1)
    idx = jnp.stack([i0, i1])
    wts = jnp.stack([w0, w1])
    vf = jnp.zeros((Bn, C), jnp.float32)
    for zi in range(2):
        for yi in range(2):
            for xi in range(2):
                corner = jax.vmap(
                    lambda f: lax.dynamic_index_in_dim(
                        lax.dynamic_index_in_dim(
                            lax.dynamic_index_in_dim(
                                f, idx[zi], axis=1, keepdims=False),
                            idx[yi], axis=1, keepdims=False),
                        idx[xi], axis=1, keepdims=False))(features)
                vf = vf + corner * (wts[zi] * wts[yi] * wts[xi])
    return vf


def _pad_cols(a, n):
    return jnp.pad(a, ((0, 0), (0, n - a.shape[1])))


def _pad_rows(a, n):
    return jnp.pad(a, ((0, n - a.shape[0]), (0, 0)))


def kernel(features, vertices, faces, enc_W1, enc_b1, enc_W2, enc_b2,
           g1_W, g1_b, g2_W, g2_b, g3_W, g3_b, g4_W, g4_b, g5_W, g5_b,
           g6_W, g6_b, h1_W, h1_b, h2_W, h2_b, h3_W, h3_b):
    f32 = jnp.float32

    e = jnp.concatenate([faces[:, 0:2], faces[:, 1:3],
                         faces[:, jnp.array([0, 2])]], axis=0)
    key = e[:, 0] * jnp.int32(N_V) + e[:, 1]
    order = jnp.argsort(key)
    es = e[order]
    ks = key[order]
    first = jnp.concatenate([jnp.ones((1,), jnp.bool_), ks[1:] != ks[:-1]])
    emask = first.astype(f32)
    src, dst = es[:, 0], es[:, 1]
    order2 = jnp.argsort(dst)
    src_d = src[order2]
    dst_d = dst[order2]
    m_d = emask[order2]
    off = jnp.searchsorted(dst_d, jnp.arange(N_V + 1, dtype=jnp.int32),
                           side='left').astype(jnp.int32)
    cm = jnp.concatenate([jnp.zeros((1,), f32), jnp.cumsum(m_d)])
    deg = (cm[off[1:]] - cm[off[:-1]]) + 1.0
    dis_n = lax.rsqrt(deg)

    dis = jnp.zeros((M_ROWS,), f32)
    dis = dis.at[0:N_V].set(dis_n).at[NP:NP + N_V].set(dis_n)
    dis128 = jnp.broadcast_to(dis[:, None], (M_ROWS, 128))

    srcg = jnp.where(m_d > 0, src_d, ZROW)
    src2 = jnp.concatenate([
        srcg, srcg + NP,
        jnp.full((E_PAD - NB * N_E,), ZROW, jnp.int32)]).astype(jnp.int32)
    pad_idx = jnp.zeros((NP - N_V,), jnp.int32)
    idxES = jnp.concatenate([
        off[1:], pad_idx, N_E + off[1:], pad_idx,
        off[:-1], pad_idx, N_E + off[:-1], pad_idx,
    ]).astype(jnp.int32)

    vf = _trilinear_const(features)
    xb = jnp.concatenate(
        [jnp.broadcast_to(vertices[None], (NB, N_V, 3)),
         jnp.broadcast_to(vf[:, None, :], (NB, N_V, vf.shape[1]))], axis=-1)
    x0 = jnp.zeros((NB, NP, 256), f32)
    x0 = x0.at[:, :N_V, :3 + vf.shape[1]].set(xb).reshape(M_ROWS, 256)

    x = _mm_bias_act(x0, _pad_rows(enc_W1, 256),
                     jnp.broadcast_to(enc_b1[None], (8, 256)), 'relu')
    x = _mm_bias_act(x, enc_W2,
                     jnp.broadcast_to(enc_b2[None], (8, 128)), 'relu')

    for W, b in ((g1_W, g1_b), (g2_W, g2_b), (g3_W, g3_b),
                 (g4_W, g4_b), (g5_W, g5_b), (g6_W, g6_b)):
        fo = W.shape[1]
        y = _mm_y(x, W, dis128)
        z = _gather_rows(y, src2)
        csx = _cumsum_excl(z)
        gES = _gather_rows(csx, idxES)
        x = _combine(gES, y, dis128,
                     jnp.broadcast_to(b[None], (8, fo)))

    x = _mm_bias_act(x, h1_W, jnp.broadcast_to(h1_b[None], (8, 128)), 'relu')
    x = _mm_bias_act(x, _pad_cols(h2_W, 128),
                     jnp.broadcast_to(jnp.pad(h2_b, (0, 64))[None], (8, 128)),
                     'relu')
    x = _mm_bias_act(x, _pad_cols(_pad_rows(h3_W, 128), 128),
                     jnp.broadcast_to(jnp.pad(h3_b, (0, 125))[None], (8, 128)),
                     'tanh')

    disp = jnp.stack([x[0:N_V, 0:3], x[NP:NP + N_V, 0:3]])
    disp = jnp.nan_to_num(disp, nan=0.0)
    return vertices + jnp.clip(disp, -2.5, 2.5)


# speedup vs baseline: 2.3951x; 1.0887x over previous
import functools

import jax
import jax.numpy as jnp
from jax import lax
from jax.experimental import pallas as pl
from jax.experimental.pallas import tpu as pltpu
from jax.experimental.pallas import tpu_sc as plsc

N_V = 10000
NP = 10240
NB = 2
M_ROWS = NB * NP
N_E = 60000
E_PAD = 122880
ZROW = N_V


def _mm_y(x, w, dis128):
    M, K = x.shape
    Nout = w.shape[1]
    BM, BN = 256, min(256, Nout)
    def body(x_ref, w_ref, d_ref, o_ref):
        acc = jnp.dot(x_ref[...], w_ref[...],
                      preferred_element_type=jnp.float32)
        o_ref[...] = acc * d_ref[:, 0:1]
    return pl.pallas_call(
        body,
        grid=(M // BM, Nout // BN),
        in_specs=[pl.BlockSpec((BM, K), lambda i, j: (i, 0)),
                  pl.BlockSpec((K, BN), lambda i, j: (0, j)),
                  pl.BlockSpec((BM, 128), lambda i, j: (i, 0))],
        out_specs=pl.BlockSpec((BM, BN), lambda i, j: (i, j)),
        out_shape=jax.ShapeDtypeStruct((M, Nout), jnp.float32),
    )(x, w, dis128)


def _mm_bias_act(x, w, b8, act):
    M, K = x.shape
    Nout = w.shape[1]
    BM, BN = 256, 128
    def body(x_ref, w_ref, b_ref, o_ref):
        acc = jnp.dot(x_ref[...], w_ref[...],
                      preferred_element_type=jnp.float32)
        acc = acc + b_ref[0:1, :]
        if act == 'relu':
            acc = jnp.maximum(acc, 0.0)
        elif act == 'tanh':
            acc = jnp.tanh(acc)
        o_ref[...] = acc
    return pl.pallas_call(
        body,
        grid=(M // BM, Nout // BN),
        in_specs=[pl.BlockSpec((BM, K), lambda i, j: (i, 0)),
                  pl.BlockSpec((K, BN), lambda i, j: (0, j)),
                  pl.BlockSpec((8, BN), lambda i, j: (0, j))],
        out_specs=pl.BlockSpec((BM, BN), lambda i, j: (i, j)),
        out_shape=jax.ShapeDtypeStruct((M, Nout), jnp.float32),
    )(x, w, b8)


def _cumsum_excl(z):
    E, D = z.shape
    RB = 1024
    CB = min(D, 256)
    def body(z_ref, o_ref, carry):
        r = pl.program_id(1)
        @pl.when(r == 0)
        def _():
            carry[...] = jnp.zeros_like(carry)
        ics = z_ref[...]
        sh = 1
        while sh < RB:
            ics = ics + jnp.concatenate(
                [jnp.zeros((sh, CB), jnp.float32), ics[:-sh]], axis=0)
            sh *= 2
        c = carry[0:1, :]
        o_ref[...] = c + jnp.concatenate(
            [jnp.zeros((1, CB), jnp.float32), ics[:-1]], axis=0)
        carry[0:1, :] = c + ics[-1:]
    return pl.pallas_call(
        body,
        grid=(D // CB, E // RB),
        in_specs=[pl.BlockSpec((RB, CB), lambda j, r: (r, j))],
        out_specs=pl.BlockSpec((RB, CB), lambda j, r: (r, j)),
        out_shape=jax.ShapeDtypeStruct((E, D), jnp.float32),
        scratch_shapes=[pltpu.VMEM((8, CB), jnp.float32)],
        compiler_params=pltpu.CompilerParams(
            dimension_semantics=("arbitrary", "arbitrary")),
    )(z)


def _combine(gES, y, dis128, b8, act='relu'):
    M, D = y.shape
    BM = 256
    CB = min(D, 256)
    nb = M // BM
    def body(e_ref, s_ref, y_ref, d_ref, b_ref, o_ref):
        agg = e_ref[...] - s_ref[...] + y_ref[...]
        acc = d_ref[:, 0:1] * agg + b_ref[0:1, :]
        if act == 'relu':
            acc = jnp.maximum(acc, 0.0)
        o_ref[...] = acc
    return pl.pallas_call(
        body,
        grid=(nb, D // CB),
        in_specs=[pl.BlockSpec((BM, CB), lambda i, j: (i, j)),
                  pl.BlockSpec((BM, CB), lambda i, j: (i + nb, j)),
                  pl.BlockSpec((BM, CB), lambda i, j: (i, j)),
                  pl.BlockSpec((BM, 128), lambda i, j: (i, 0)),
                  pl.BlockSpec((8, CB), lambda i, j: (0, j))],
        out_specs=pl.BlockSpec((BM, CB), lambda i, j: (i, j)),
        out_shape=jax.ShapeDtypeStruct((M, D), jnp.float32),
    )(gES, gES, y, dis128, b8)


def _scale_rows(x, dis128):
    M, D = x.shape
    BM = 256
    CB = min(D, 256)
    def body(x_ref, d_ref, o_ref):
        o_ref[...] = x_ref[...] * d_ref[:, 0:1]
    return pl.pallas_call(
        body,
        grid=(M // BM, D // CB),
        in_specs=[pl.BlockSpec((BM, CB), lambda i, j: (i, j)),
                  pl.BlockSpec((BM, 128), lambda i, j: (i, 0))],
        out_specs=pl.BlockSpec((BM, CB), lambda i, j: (i, j)),
        out_shape=jax.ShapeDtypeStruct((M, D), jnp.float32),
    )(x, dis128)


def _gather_rows(table, idx):
    V, D = table.shape
    Bi = idx.shape[0]
    info = plsc.get_sparse_core_info()
    NW = info.num_cores * info.num_subcores
    bpw = Bi // NW
    assert Bi % NW == 0 and bpw % 8 == 0
    ch = 8
    for c in range(min(bpw, max(8, 49152 // D)), 7, -8):
        if bpw % c == 0 and (bpw // c) % 2 == 0:
            ch = c
            break
    nch = bpw // ch
    assert nch % 2 == 0
    mesh = plsc.VectorSubcoreMesh(core_axis_name="c", subcore_axis_name="s")

    @functools.partial(
        pl.kernel, mesh=mesh,
        out_type=jax.ShapeDtypeStruct((Bi, D), jnp.float32),
        scratch_types=[pltpu.VMEM((ch,), jnp.int32),
                       pltpu.VMEM((ch,), jnp.int32),
                       pltpu.VMEM((ch, D), jnp.float32),
                       pltpu.VMEM((ch, D), jnp.float32),
                       pltpu.SemaphoreType.DMA,
                       pltpu.SemaphoreType.DMA])
    def k(table_hbm, idx_hbm, out_hbm, idx0, idx1, rows0, rows1, g0, g1):
        wid = lax.axis_index("s") * info.num_cores + lax.axis_index("c")
        base0 = wid * bpw

        def start(iv, rv, sem, j):
            pltpu.sync_copy(idx_hbm.at[pl.ds(base0 + j * ch, ch)], iv)
            pltpu.async_copy(table_hbm.at[iv], rv, sem)

        def finish(iv, rv, sem, j):
            pltpu.make_async_copy(table_hbm.at[iv], rv, sem).wait()
            pltpu.sync_copy(rv, out_hbm.at[pl.ds(base0 + j * ch, ch)])

        start(idx0, rows0, g0, 0)

        def body(j2, carry):
            a = 2 * j2
            start(idx1, rows1, g1, a + 1)
            finish(idx0, rows0, g0, a)
            @pl.when(j2 + 1 < nch // 2)
            def _():
                start(idx0, rows0, g0, a + 2)
            finish(idx1, rows1, g1, a + 1)
            return carry
        lax.fori_loop(0, nch // 2, body, 0)

    return k(table, idx)


def _trilinear_const(features):
    Bn, C, D, H, W = features.shape
    g = jnp.float32(-1.0 / 1.5)
    ix = jnp.clip((g + 1.0) * 0.5 * (W - 1), 0.0, W - 1.0)
    x0 = jnp.floor(ix)
    w1 = (ix - x0).astype(jnp.float32)
    w0 = 1.0 - w1
    i0 = jnp.clip(x0.astype(jnp.int32), 0, W - 1)
    i1 = jnp.clip(i0 + 1, 0, W - 1)
    idx = jnp.stack([i0, i1])
    wts = jnp.stack([w0, w1])
    vf = jnp.zeros((Bn, C), jnp.float32)
    for zi in range(2):
        for yi in range(2):
            for xi in range(2):
                corner = jax.vmap(
                    lambda f: lax.dynamic_index_in_dim(
                        lax.dynamic_index_in_dim(
                            lax.dynamic_index_in_dim(
                                f, idx[zi], axis=1, keepdims=False),
                            idx[yi], axis=1, keepdims=False),
                        idx[xi], axis=1, keepdims=False))(features)
                vf = vf + corner * (wts[zi] * wts[yi] * wts[xi])
    return vf


def _pad_cols(a, n):
    return jnp.pad(a, ((0, 0), (0, n - a.shape[1])))


def _pad_rows(a, n):
    return jnp.pad(a, ((0, n - a.shape[0]), (0, 0)))


def kernel(features, vertices, faces, enc_W1, enc_b1, enc_W2, enc_b2,
           g1_W, g1_b, g2_W, g2_b, g3_W, g3_b, g4_W, g4_b, g5_W, g5_b,
           g6_W, g6_b, h1_W, h1_b, h2_W, h2_b, h3_W, h3_b):
    f32 = jnp.float32

    e = jnp.concatenate([faces[:, 0:2], faces[:, 1:3],
                         faces[:, jnp.array([0, 2])]], axis=0)
    key = e[:, 0] * jnp.int32(N_V) + e[:, 1]
    order = jnp.argsort(key)
    es = e[order]
    ks = key[order]
    first = jnp.concatenate([jnp.ones((1,), jnp.bool_), ks[1:] != ks[:-1]])
    emask = first.astype(f32)
    src, dst = es[:, 0], es[:, 1]
    order2 = jnp.argsort(dst)
    src_d = src[order2]
    dst_d = dst[order2]
    m_d = emask[order2]
    off = jnp.searchsorted(dst_d, jnp.arange(N_V + 1, dtype=jnp.int32),
                           side='left').astype(jnp.int32)
    cm = jnp.concatenate([jnp.zeros((1,), f32), jnp.cumsum(m_d)])
    deg = (cm[off[1:]] - cm[off[:-1]]) + 1.0
    dis_n = lax.rsqrt(deg)

    dis = jnp.zeros((M_ROWS,), f32)
    dis = dis.at[0:N_V].set(dis_n).at[NP:NP + N_V].set(dis_n)
    dis128 = jnp.broadcast_to(dis[:, None], (M_ROWS, 128))

    srcg = jnp.where(m_d > 0, src_d, ZROW)
    src2 = jnp.concatenate([
        srcg, srcg + NP,
        jnp.full((E_PAD - NB * N_E,), ZROW, jnp.int32)]).astype(jnp.int32)
    pad_idx = jnp.zeros((NP - N_V,), jnp.int32)
    idxES = jnp.concatenate([
        off[1:], pad_idx, N_E + off[1:], pad_idx,
        off[:-1], pad_idx, N_E + off[:-1], pad_idx,
    ]).astype(jnp.int32)

    vf = _trilinear_const(features)
    xb = jnp.concatenate(
        [jnp.broadcast_to(vertices[None], (NB, N_V, 3)),
         jnp.broadcast_to(vf[:, None, :], (NB, N_V, vf.shape[1]))], axis=-1)
    x0 = jnp.zeros((NB, NP, 256), f32)
    x0 = x0.at[:, :N_V, :3 + vf.shape[1]].set(xb).reshape(M_ROWS, 256)

    x = _mm_bias_act(x0, _pad_rows(enc_W1, 256),
                     jnp.broadcast_to(enc_b1[None], (8, 256)), 'relu')
    x = _mm_bias_act(x, enc_W2,
                     jnp.broadcast_to(enc_b2[None], (8, 128)), 'relu')

    for W, b in ((g1_W, g1_b), (g2_W, g2_b), (g3_W, g3_b),
                 (g4_W, g4_b), (g5_W, g5_b), (g6_W, g6_b)):
        fi, fo = W.shape
        if fi < fo:
            u = _scale_rows(x, dis128)
            z = _gather_rows(u, src2)
            csx = _cumsum_excl(z)
            gES = _gather_rows(csx, idxES)
            xa = _combine(gES, u, dis128, jnp.zeros((8, fi), f32), 'none')
            x = _mm_bias_act(xa, W, jnp.broadcast_to(b[None], (8, fo)),
                             'relu')
        else:
            y = _mm_y(x, W, dis128)
            z = _gather_rows(y, src2)
            csx = _cumsum_excl(z)
            gES = _gather_rows(csx, idxES)
            x = _combine(gES, y, dis128,
                         jnp.broadcast_to(b[None], (8, fo)), 'relu')

    x = _mm_bias_act(x, h1_W, jnp.broadcast_to(h1_b[None], (8, 128)), 'relu')
    x = _mm_bias_act(x, _pad_cols(h2_W, 128),
                     jnp.broadcast_to(jnp.pad(h2_b, (0, 64))[None], (8, 128)),
                     'relu')
    x = _mm_bias_act(x, _pad_cols(_pad_rows(h3_W, 128), 128),
                     jnp.broadcast_to(jnp.pad(h3_b, (0, 125))[None], (8, 128)),
                     'tanh')

    disp = jnp.stack([x[0:N_V, 0:3], x[NP:NP + N_V, 0:3]])
    disp = jnp.nan_to_num(disp, nan=0.0)
    return vertices + jnp.clip(disp, -2.5, 2.5)
